# baseline (speedup 1.0000x reference)
import functools

import jax
import jax.numpy as jnp
from jax import lax
from jax.experimental import pallas as pl
from jax.experimental.pallas import tpu as pltpu
from jax.experimental.pallas import tpu_sc as plsc

NUM_EMB = 1000000
DIM = 32
BATCH = 16384
HIST = 20
TOTAL = BATCH * HIST

_info = plsc.get_sparse_core_info()
_NC = _info.num_cores
_NS = _info.num_subcores
_NW = _NC * _NS
_L = _info.num_lanes

_B_PER_W = TOTAL // _NW
_CHUNK = 256
_NCHUNKS = _B_PER_W // _CHUNK
_GBUF = 3
_TBUF = 2
_TSTR = _CHUNK + 1
_PDIM = 128


def _body(idx_hbm, table_hbm, out_hbm, idx_v, rows_v, t_v, *sems):
    g_sems = sems[:_GBUF]
    o_sems = sems[_GBUF:]
    wid = lax.axis_index("s") * _NC + lax.axis_index("c")
    base = wid * _B_PER_W
    pltpu.sync_copy(idx_hbm.at[pl.ds(base, _B_PER_W)], idx_v)

    iota = lax.iota(jnp.int32, _L)
    d_lo = iota
    d_hi = iota + _L

    def start_gather(c):
        g = c % _GBUF
        idx_sl = idx_v.at[pl.ds(c * _CHUNK, _CHUNK)]
        return pltpu.async_copy(table_hbm.at[idx_sl], rows_v.at[g], g_sems[g])

    def start_out(c):
        t = c % _TBUF
        j0 = base + c * _CHUNK
        h = j0 // BATCH
        b0 = j0 % BATCH
        return pltpu.async_copy(
            t_v.at[t, :, pl.ds(0, _CHUNK)],
            out_hbm.at[h, :, pl.ds(b0, _CHUNK)],
            o_sems[t],
        )

    gathers = [start_gather(c) for c in range(min(_GBUF, _NCHUNKS))]
    gathers += [None] * (_NCHUNKS - len(gathers))
    outs = [None] * _NCHUNKS
    for c in range(_NCHUNKS):
        g = c % _GBUF
        t = c % _TBUF
        gathers[c].wait()
        if c >= _TBUF:
            outs[c - _TBUF].wait()

        def transpose_one(l, _, g=g, t=t):
            x0 = rows_v[g, l, pl.ds(0, _L)]
            x1 = rows_v[g, l, pl.ds(_L, _L)]
            lv = jnp.broadcast_to(l, (_L,))
            tv = jnp.broadcast_to(t, (_L,))
            plsc.store_scatter(t_v, [tv, d_lo, lv], x0)
            plsc.store_scatter(t_v, [tv, d_hi, lv], x1)
            return _

        lax.fori_loop(0, _CHUNK, transpose_one, 0, unroll=8)
        outs[c] = start_out(c)
        nc = c + _GBUF
        if nc < _NCHUNKS:
            gathers[nc] = start_gather(nc)
    for c in range(_NCHUNKS - _TBUF, _NCHUNKS):
        outs[c].wait()


_gather = pl.kernel(
    _body,
    out_type=jax.ShapeDtypeStruct((HIST, DIM, BATCH), jnp.float32),
    mesh=plsc.VectorSubcoreMesh(core_axis_name="c", subcore_axis_name="s"),
    scratch_types=[
        pltpu.VMEM((_B_PER_W,), jnp.int32),
        pltpu.VMEM((_GBUF, _CHUNK, _PDIM), jnp.float32),
        pltpu.VMEM((_TBUF, DIM, _TSTR), jnp.float32),
    ]
    + [pltpu.SemaphoreType.DMA] * (_GBUF + _TBUF),
    compiler_params=pltpu.CompilerParams(
        use_tc_tiling_on_sc=False, needs_layout_passes=False
    ),
)


@jax.jit
def kernel(token_ids, weights):
    idx = jnp.reshape(token_ids.T, (TOTAL,)).astype(jnp.int32)
    w_pad = jnp.pad(weights, ((0, 0), (0, _PDIM - DIM)))
    out_t = _gather(idx, w_pad)
    return jnp.transpose(out_t, (2, 0, 1))
